# Initial kernel scaffold; baseline (speedup 1.0000x reference)
import jax
import jax.numpy as jnp
from jax.experimental import pallas as pl


def kernel(z, edge_index):
    raise NotImplementedError("write your pallas kernel here")


# speedup vs baseline: 3.5637x; 3.5637x over previous
import functools

import jax
import jax.numpy as jnp
from jax import lax
from jax.experimental import pallas as pl
from jax.experimental.pallas import tpu as pltpu
from jax.experimental.pallas import tpu_sc as plsc

N_NODES = 10000
D = 128
N_EDGES = 320000
NC = 2
NS = 16
NW = NC * NS
EPW = N_EDGES // NW
C = 80
N_CHUNKS = EPW // C


def _normalize_body(z_ref, o_ref):
    x = z_ref[...]
    ss = jnp.sum(x * x, axis=1, keepdims=True)
    o_ref[...] = x * lax.rsqrt(ss)


def _normalize(z):
    return pl.pallas_call(
        _normalize_body,
        out_shape=jax.ShapeDtypeStruct((N_NODES, D), jnp.float32),
    )(z)


_SC_MESH = plsc.VectorSubcoreMesh(core_axis_name="c", subcore_axis_name="s")


@functools.partial(
    pl.kernel,
    out_type=jax.ShapeDtypeStruct((N_EDGES,), jnp.float32),
    mesh=_SC_MESH,
    scratch_types=[
        pltpu.VMEM((EPW,), jnp.int32),
        pltpu.VMEM((EPW,), jnp.int32),
        pltpu.VMEM((C, D), jnp.float32),
        pltpu.VMEM((C, D), jnp.float32),
        pltpu.VMEM((C,), jnp.float32),
        pltpu.SemaphoreType.DMA,
        pltpu.SemaphoreType.DMA,
    ],
)
def _sc_edge_dot(zn_hbm, src_hbm, dst_hbm, out_hbm,
                 idx_s, idx_d, rows_s, rows_d, out_v, sem_s, sem_d):
    wid = lax.axis_index("s") * NC + lax.axis_index("c")
    wbase = wid * EPW
    pltpu.sync_copy(src_hbm.at[pl.ds(wbase, EPW)], idx_s)
    pltpu.sync_copy(dst_hbm.at[pl.ds(wbase, EPW)], idx_d)

    def chunk_body(i, _):
        off = i * C
        cs = pltpu.async_copy(zn_hbm.at[idx_s.at[pl.ds(off, C)]], rows_s, sem_s)
        cd = pltpu.async_copy(zn_hbm.at[idx_d.at[pl.ds(off, C)]], rows_d, sem_d)
        cs.wait()
        cd.wait()

        lane = lax.iota(jnp.int32, 16)
        rots = [(lane + r) % 16 for r in (8, 4, 2, 1)]
        _dnums = lax.GatherDimensionNumbers(
            offset_dims=(), collapsed_slice_dims=(0,), start_index_map=(0,))

        def _permute(x, idx):
            return lax.gather(x, idx[:, None], _dnums, slice_sizes=(1,),
                              mode=lax.GatherScatterMode.PROMISE_IN_BOUNDS)

        def group_body(k, _):
            base = k * 16
            vec = jnp.zeros((16,), jnp.float32)
            for g in range(16):
                e = base + g
                acc = rows_s[e, pl.ds(0, 16)] * rows_d[e, pl.ds(0, 16)]
                for j in range(1, D // 16):
                    acc = acc + rows_s[e, pl.ds(j * 16, 16)] * rows_d[e, pl.ds(j * 16, 16)]
                for rot in rots:
                    acc = acc + _permute(acc, rot)
                vec = jnp.where(lane == g, acc * 0.5 + 0.5, vec)
            out_v[pl.ds(base, 16)] = vec
            return 0

        lax.fori_loop(0, C // 16, group_body, 0)
        pltpu.sync_copy(out_v, out_hbm.at[pl.ds(wbase + off, C)])
        return 0

    lax.fori_loop(0, N_CHUNKS, chunk_body, 0)


def kernel(z, edge_index):
    zn = _normalize(z)
    ei = edge_index.astype(jnp.int32)
    return _sc_edge_dot(zn, ei[0], ei[1])


# speedup vs baseline: 5.0028x; 1.4038x over previous
import functools

import jax
import jax.numpy as jnp
from jax import lax
from jax.experimental import pallas as pl
from jax.experimental.pallas import tpu as pltpu
from jax.experimental.pallas import tpu_sc as plsc

N_NODES = 10000
D = 128
N_EDGES = 320000
NC = 2
NS = 16
NW = NC * NS
EPW = N_EDGES // NW
C = 80
N_CHUNKS = EPW // C


def _normalize_body(z_ref, o_ref):
    x = z_ref[...]
    ss = jnp.sum(x * x, axis=1, keepdims=True)
    o_ref[...] = x * lax.rsqrt(ss)


def _normalize(z):
    return pl.pallas_call(
        _normalize_body,
        out_shape=jax.ShapeDtypeStruct((N_NODES, D), jnp.float32),
    )(z)


_SC_MESH = plsc.VectorSubcoreMesh(core_axis_name="c", subcore_axis_name="s")


@functools.partial(
    pl.kernel,
    out_type=jax.ShapeDtypeStruct((N_EDGES,), jnp.float32),
    mesh=_SC_MESH,
    scratch_types=[
        pltpu.VMEM((EPW,), jnp.int32),
        pltpu.VMEM((EPW,), jnp.int32),
        pltpu.VMEM((2, C, D), jnp.float32),
        pltpu.VMEM((2, C, D), jnp.float32),
        pltpu.VMEM((C,), jnp.float32),
        pltpu.SemaphoreType.DMA,
        pltpu.SemaphoreType.DMA,
        pltpu.SemaphoreType.DMA,
        pltpu.SemaphoreType.DMA,
    ],
)
def _sc_edge_dot(zn_hbm, src_hbm, dst_hbm, out_hbm,
                 idx_s, idx_d, rows_s, rows_d, out_v,
                 sem_s0, sem_d0, sem_s1, sem_d1):
    wid = lax.axis_index("s") * NC + lax.axis_index("c")
    wbase = wid * EPW
    pltpu.sync_copy(src_hbm.at[pl.ds(wbase, EPW)], idx_s)
    pltpu.sync_copy(dst_hbm.at[pl.ds(wbase, EPW)], idx_d)

    sems = ((sem_s0, sem_d0), (sem_s1, sem_d1))

    def issue(buf, i):
        off = i * C
        sem_s, sem_d = sems[buf]
        pltpu.async_copy(
            zn_hbm.at[idx_s.at[pl.ds(off, C)]], rows_s.at[buf], sem_s)
        pltpu.async_copy(
            zn_hbm.at[idx_d.at[pl.ds(off, C)]], rows_d.at[buf], sem_d)

    def wait(buf):
        sem_s, sem_d = sems[buf]
        pltpu.make_async_copy(
            zn_hbm.at[idx_s.at[pl.ds(0, C)]], rows_s.at[buf], sem_s).wait()
        pltpu.make_async_copy(
            zn_hbm.at[idx_d.at[pl.ds(0, C)]], rows_d.at[buf], sem_d).wait()

    lane = lax.iota(jnp.int32, 16)
    rots = [(lane + r) % 16 for r in (8, 4, 2, 1)]
    _dnums = lax.GatherDimensionNumbers(
        offset_dims=(), collapsed_slice_dims=(0,), start_index_map=(0,))

    def _permute(x, idx):
        return lax.gather(x, idx[:, None], _dnums, slice_sizes=(1,),
                          mode=lax.GatherScatterMode.PROMISE_IN_BOUNDS)

    def group_loop(buf, out_off):
        rs = rows_s.at[buf]
        rd = rows_d.at[buf]

        def group_body(k, _):
            base = k * 16
            vec = jnp.zeros((16,), jnp.float32)
            for g in range(16):
                e = base + g
                acc = rs[e, pl.ds(0, 16)] * rd[e, pl.ds(0, 16)]
                for j in range(1, D // 16):
                    acc = acc + rs[e, pl.ds(j * 16, 16)] * rd[e, pl.ds(j * 16, 16)]
                for rot in rots:
                    acc = acc + _permute(acc, rot)
                vec = jnp.where(lane == g, acc * 0.5 + 0.5, vec)
            out_v[pl.ds(base, 16)] = vec
            return 0

        lax.fori_loop(0, C // 16, group_body, 0)
        pltpu.sync_copy(out_v, out_hbm.at[pl.ds(wbase + out_off, C)])

    issue(0, 0)

    def pair_body(p, _):
        i0 = 2 * p
        issue(1, i0 + 1)
        wait(0)
        group_loop(0, i0 * C)
        issue(0, i0 + 2)
        wait(1)
        group_loop(1, (i0 + 1) * C)
        return 0

    lax.fori_loop(0, (N_CHUNKS - 1) // 2, pair_body, 0)
    wait(0)
    group_loop(0, (N_CHUNKS - 1) * C)


def kernel(z, edge_index):
    zn = _normalize(z)
    ei = edge_index.astype(jnp.int32)
    return _sc_edge_dot(zn, ei[0], ei[1])
